# baseline (speedup 1.0000x reference)
import functools

import jax
import jax.numpy as jnp
from jax.experimental import pallas as pl
from jax.experimental.pallas import tpu as pltpu


def _bce_batch_kernel(x_ref, lab_ref, loss_ref, cnt_ref, *, num_classes,
                      ignore_index):
    x = x_ref[...]
    lab = lab_ref[...]
    c_idx = jax.lax.broadcasted_iota(jnp.int32, x.shape, 0)
    valid = jnp.logical_and(lab != ignore_index,
                            lab < num_classes).astype(jnp.float32)
    sp = jnp.maximum(x, 0.0) + jnp.log1p(jnp.exp(-jnp.abs(x)))
    sel = jnp.where(c_idx == lab, x, 0.0)
    pix = jnp.sum(sp - sel, axis=0, keepdims=True)
    loss_ref[...] = jnp.sum(pix * valid, keepdims=True)
    cnt_ref[...] = jnp.sum(valid, keepdims=True)


def kernel(inputs, targets, *, ignore_index=255):
    B, C, H, W = inputs.shape
    HW = H * W

    x = inputs.reshape(B, C, HW)
    if x.dtype != jnp.float32:
        x = x.astype(jnp.float32)
    lab = targets.reshape(B, 1, HW)
    if lab.dtype != jnp.int32:
        lab = lab.astype(jnp.int32)

    kern = functools.partial(_bce_batch_kernel, num_classes=C,
                             ignore_index=ignore_index)
    loss_parts, cnt_parts = pl.pallas_call(
        kern,
        grid=(B,),
        in_specs=[pl.BlockSpec((None, C, HW), lambda b: (b, 0, 0)),
                  pl.BlockSpec((None, 1, HW), lambda b: (b, 0, 0))],
        out_specs=[pl.BlockSpec((None, 1, 1), lambda b: (b, 0, 0)),
                   pl.BlockSpec((None, 1, 1), lambda b: (b, 0, 0))],
        out_shape=(jax.ShapeDtypeStruct((B, 1, 1), jnp.float32),
                   jax.ShapeDtypeStruct((B, 1, 1), jnp.float32)),
        compiler_params=pltpu.CompilerParams(
            dimension_semantics=("parallel",)),
    )(x, lab)

    return jnp.sum(loss_parts) / jnp.sum(cnt_parts)
